# baseline (speedup 1.0000x reference)
import dataclasses
import functools

import jax
import jax.numpy as jnp
from jax import lax
from jax.experimental import pallas as pl
from jax.experimental.pallas import tpu as pltpu
from jax.experimental.pallas import tpu_sc as plsc

N = 10000
E = 320000
F_IN = 128
F_OUT = 64
F2 = 2 * F_OUT

NC = 2
NS = 16
NP = 10240
RED = NP // NS
EPT = E // NS
CH = 128
ROWS_PT = NP // NS
ZB = 128

_mesh = plsc.VectorSubcoreMesh(core_axis_name="c", subcore_axis_name="s")

_sc_params = pltpu.CompilerParams()
if "needs_layout_passes" in pltpu.CompilerParams.__dataclass_fields__:
    _sc_params = dataclasses.replace(_sc_params, needs_layout_passes=False)


def _deg_body(row_hbm, col_hbm, ew_hbm, dout_hbm, din_hbm,
              idx_v, w_v, deg_v, res_v, red_v, stage_sh):
    cid = lax.axis_index("c")
    sid = lax.axis_index("s")
    base = sid * EPT

    @pl.when(cid == 0)
    def _():
        pltpu.sync_copy(row_hbm.at[pl.ds(base, EPT)], idx_v)

    @pl.when(cid == 1)
    def _():
        pltpu.sync_copy(col_hbm.at[pl.ds(base, EPT)], idx_v)

    pltpu.sync_copy(ew_hbm.at[pl.ds(base, EPT)], w_v)

    @pl.loop(0, NP, step=16)
    def _(i):
        deg_v[pl.ds(i, 16)] = jnp.zeros((16,), jnp.float32)

    @pl.loop(0, EPT, step=16)
    def _(j):
        plsc.addupdate_scatter(deg_v, [idx_v[pl.ds(j, 16)]], w_v[pl.ds(j, 16)])

    pltpu.sync_copy(deg_v, stage_sh.at[sid])
    plsc.subcore_barrier()
    pltpu.sync_copy(stage_sh.at[:, pl.ds(sid * RED, RED)], red_v)

    @pl.loop(0, RED, step=16)
    def _(i):
        acc = red_v[0, pl.ds(i, 16)]
        for r in range(1, NS):
            acc = acc + red_v[r, pl.ds(i, 16)]
        res_v[pl.ds(i, 16)] = acc

    @pl.when(cid == 0)
    def _():
        pltpu.sync_copy(res_v, dout_hbm.at[pl.ds(sid * RED, RED)])

    @pl.when(cid == 1)
    def _():
        pltpu.sync_copy(res_v, din_hbm.at[pl.ds(sid * RED, RED)])


_deg_kernel = functools.partial(
    pl.kernel,
    out_type=(jax.ShapeDtypeStruct((NP,), jnp.float32),
              jax.ShapeDtypeStruct((NP,), jnp.float32)),
    mesh=_mesh,
    scratch_types=[
        pltpu.VMEM((EPT,), jnp.int32),
        pltpu.VMEM((EPT,), jnp.float32),
        pltpu.VMEM((NP,), jnp.float32),
        pltpu.VMEM((RED,), jnp.float32),
        pltpu.VMEM((NS, RED), jnp.float32),
        pltpu.VMEM_SHARED((NS, NP), jnp.float32),
    ],
    compiler_params=_sc_params,
)(_deg_body)


NCH_STD = 160
NCH_LAST = 100
ER = E // CH
CPI = 8


def _prop_half(rc_hbm, tbl_hbm, acc_sh, rc_i, gbufs, isems, gsems, ssems,
               r0, nch):
    def i_start(j, p):
        pltpu.async_copy(rc_hbm.at[r0 + j], rc_i.at[p], isems[p])

    def i_wait(p):
        pltpu.make_async_copy(rc_hbm.at[r0], rc_i.at[p], isems[p]).wait()

    def g_start(p):
        pltpu.async_copy(tbl_hbm.at[rc_i.at[p, 0]], gbufs[p], gsems[p])

    def g_wait(p):
        pltpu.make_async_copy(tbl_hbm.at[rc_i.at[p, 0]], gbufs[p],
                              gsems[p]).wait()

    def s_start(p):
        pltpu.async_copy(gbufs[p], acc_sh.at[rc_i.at[p, 1]], ssems[p],
                         add=True)

    def s_wait(p):
        pltpu.make_async_copy(gbufs[p], acc_sh.at[rc_i.at[p, 1]],
                              ssems[p]).wait()

    def pair(a, prefetch, terminal):
        g_wait(0)
        s_start(0)
        g_wait(1)
        s_start(1)
        if prefetch:
            s_wait(0)
            i_start(a + 2, 0)
            s_wait(1)
            i_start(a + 3, 1)
            i_wait(0)
            g_start(0)
            i_wait(1)
            g_start(1)
        if terminal:
            s_wait(0)
            s_wait(1)

    i_start(0, 0)
    i_start(1, 1)
    i_wait(0)
    g_start(0)
    i_wait(1)
    g_start(1)

    full = (nch - CPI) // CPI

    @pl.loop(0, full)
    def _(k):
        c = CPI * k
        for t in range(CPI // 2):
            pair(c + 2 * t, True, False)

    a = full * CPI
    while a < nch:
        pair(a, a + 2 < nch, a + 2 >= nch)
        a += 2


def _prop_body(rc_hbm, tout_hbm, tin_hbm, pout_hbm, pin_hbm,
               rc_i, g_0, g_1, acc_sh,
               sem_i0, sem_i1, sem_g0, sem_g1, sem_s0, sem_s1):
    cid = lax.axis_index("c")
    sid = lax.axis_index("s")
    gbufs = (g_0, g_1)
    isems = (sem_i0, sem_i1)
    gsems = (sem_g0, sem_g1)
    ssems = (sem_s0, sem_s1)

    @pl.loop(0, ZB)
    def _(i):
        @pl.loop(0, F2, step=16)
        def _(k2):
            g_0[i, pl.ds(k2, 16)] = jnp.zeros((16,), jnp.float32)

    for k in range(ROWS_PT // ZB):
        pltpu.sync_copy(g_0, acc_sh.at[pl.ds(sid * ROWS_PT + k * ZB, ZB)])
    plsc.subcore_barrier()

    def run(tbl_hbm):
        @pl.when(sid < NS - 1)
        def _():
            _prop_half(rc_hbm, tbl_hbm, acc_sh, rc_i, gbufs, isems, gsems,
                       ssems, sid * NCH_STD, NCH_STD)

        @pl.when(sid == NS - 1)
        def _():
            _prop_half(rc_hbm, tbl_hbm, acc_sh, rc_i, gbufs, isems, gsems,
                       ssems, (NS - 1) * NCH_STD, NCH_LAST)

    @pl.when(cid == 0)
    def _():
        run(tout_hbm)

    @pl.when(cid == 1)
    def _():
        run(tin_hbm)

    plsc.subcore_barrier()

    @pl.when(cid == 0)
    def _():
        pltpu.sync_copy(acc_sh.at[pl.ds(sid * ROWS_PT, ROWS_PT)],
                        pout_hbm.at[pl.ds(sid * ROWS_PT, ROWS_PT)])

    @pl.when(cid == 1)
    def _():
        pltpu.sync_copy(acc_sh.at[pl.ds(sid * ROWS_PT, ROWS_PT)],
                        pin_hbm.at[pl.ds(sid * ROWS_PT, ROWS_PT)])


_prop_kernel = functools.partial(
    pl.kernel,
    out_type=(jax.ShapeDtypeStruct((NP, F2), jnp.float32),
              jax.ShapeDtypeStruct((NP, F2), jnp.float32)),
    mesh=_mesh,
    scratch_types=[
        pltpu.VMEM((2, 2, CH), jnp.int32),
        pltpu.VMEM((CH, F2), jnp.float32),
        pltpu.VMEM((CH, F2), jnp.float32),
        pltpu.VMEM_SHARED((NP, F2), jnp.float32),
    ] + [pltpu.SemaphoreType.DMA] * 6,
    compiler_params=_sc_params,
)(_prop_body)


BLK = 1000


def _mm_body(x_ref, w_ref, o_ref):
    o_ref[...] = jnp.dot(x_ref[...], w_ref[...],
                         preferred_element_type=jnp.float32)


def _matmul_tc(x, w):
    m, k = x.shape
    n = w.shape[1]
    return pl.pallas_call(
        _mm_body,
        grid=(m // BLK,),
        in_specs=[pl.BlockSpec((BLK, k), lambda i: (i, 0)),
                  pl.BlockSpec((k, n), lambda i: (0, 0))],
        out_specs=pl.BlockSpec((BLK, n), lambda i: (i, 0)),
        out_shape=jax.ShapeDtypeStruct((m, n), jnp.float32),
    )(x, w)


def _scale_body(u_ref, d_ref, o_ref):
    d = d_ref[...]
    r = jnp.where(d > 0, 1.0 / d, 0.0)
    o_ref[...] = u_ref[...] * r


def _scale_tc(u, d):
    m, n = u.shape
    return pl.pallas_call(
        _scale_body,
        grid=(m // BLK,),
        in_specs=[pl.BlockSpec((BLK, n), lambda i: (i, 0)),
                  pl.BlockSpec((BLK, 1), lambda i: (i, 0))],
        out_specs=pl.BlockSpec((BLK, n), lambda i: (i, 0)),
        out_shape=jax.ShapeDtypeStruct((m, n), jnp.float32),
    )(u, d)


def _combine_body(b0_ref, po_ref, pi_ref, din_ref, bz_ref, bh_ref, lw_ref,
                  lb_ref, o_ref):
    d = din_ref[...]
    r = jnp.where(d > 0, 1.0 / d, 0.0)
    b0 = b0_ref[...]
    po = po_ref[...]
    pi = pi_ref[...] * r
    lz = b0[:, :F_OUT] + po[:, :F_OUT] + pi[:, :F_OUT] + bz_ref[...]
    lh = b0[:, F_OUT:] + po[:, F_OUT:] + pi[:, F_OUT:] + bh_ref[...]
    z = jax.nn.sigmoid(lz)
    ht = jnp.tanh(lh)
    h = jax.nn.relu((1.0 - z) * ht)
    o_ref[...] = (jnp.dot(h, lw_ref[...], preferred_element_type=jnp.float32)
                  + lb_ref[...])


def _combine_tc(b0, po, pi, din, bz2, bh2, lw, lb2):
    m = b0.shape[0]
    return pl.pallas_call(
        _combine_body,
        grid=(m // BLK,),
        in_specs=[pl.BlockSpec((BLK, F2), lambda i: (i, 0)),
                  pl.BlockSpec((BLK, F2), lambda i: (i, 0)),
                  pl.BlockSpec((BLK, F2), lambda i: (i, 0)),
                  pl.BlockSpec((BLK, 1), lambda i: (i, 0)),
                  pl.BlockSpec((1, F_OUT), lambda i: (0, 0)),
                  pl.BlockSpec((1, F_OUT), lambda i: (0, 0)),
                  pl.BlockSpec((F_OUT, 1), lambda i: (0, 0)),
                  pl.BlockSpec((1, 1), lambda i: (0, 0))],
        out_specs=pl.BlockSpec((BLK, 1), lambda i: (i, 0)),
        out_shape=jax.ShapeDtypeStruct((m, 1), jnp.float32),
    )(b0, po, pi, din, bz2, bh2, lw, lb2)


def kernel(x, edge_index, edge_weight, h_, c, Wz, bz, Wr, br, Wh, bh,
           lin_w, lin_b):
    wo = jnp.concatenate([Wz[0, 1, :F_IN], Wh[0, 1, :F_IN]], axis=1)
    wi = jnp.concatenate([Wz[1, 1, :F_IN], Wh[1, 1, :F_IN]], axis=1)
    w0 = jnp.concatenate([Wz[0, 0, :F_IN] + Wz[1, 0, :F_IN],
                          Wh[0, 0, :F_IN] + Wh[1, 0, :F_IN]], axis=1)
    wall = jnp.concatenate([wo, wi, w0], axis=1)

    row = edge_index[0]
    col = edge_index[1]
    u = _matmul_tc(x, wall)
    deg_out_p, deg_in_p = _deg_kernel(row, col, edge_weight)
    t_out = _scale_tc(u[:, :F2], deg_out_p[:N].reshape(N, 1))
    t_in = u[:, F2:2 * F2]
    rc = jnp.stack([row.reshape(ER, CH), col.reshape(ER, CH)], axis=1)
    p_out_p, p_in_p = _prop_kernel(rc, t_out, t_in)
    p_out = p_out_p[:N]
    p_in = p_in_p[:N]
    out = _combine_tc(u[:, 2 * F2:], p_out, p_in,
                      deg_in_p[:N].reshape(N, 1),
                      bz.reshape(1, F_OUT), bh.reshape(1, F_OUT),
                      lin_w, lin_b.reshape(1, 1))
    return out


# speedup vs baseline: 1.0171x; 1.0171x over previous
import dataclasses
import functools

import jax
import jax.numpy as jnp
from jax import lax
from jax.experimental import pallas as pl
from jax.experimental.pallas import tpu as pltpu
from jax.experimental.pallas import tpu_sc as plsc

N = 10000
E = 320000
F_IN = 128
F_OUT = 64
F2 = 2 * F_OUT

NC = 2
NS = 16
NP = 10240
RED = NP // NS
EPT = E // NS
CH = 128
ROWS_PT = NP // NS
ZB = 128

_mesh = plsc.VectorSubcoreMesh(core_axis_name="c", subcore_axis_name="s")

_sc_params = pltpu.CompilerParams()
if "needs_layout_passes" in pltpu.CompilerParams.__dataclass_fields__:
    _sc_params = dataclasses.replace(_sc_params, needs_layout_passes=False)


def _deg_body(row_hbm, col_hbm, ew_hbm, dout_hbm, din_hbm,
              idx_v, w_v, deg_v, res_v, red_v, stage_sh):
    cid = lax.axis_index("c")
    sid = lax.axis_index("s")
    base = sid * EPT

    @pl.when(cid == 0)
    def _():
        pltpu.sync_copy(row_hbm.at[pl.ds(base, EPT)], idx_v)

    @pl.when(cid == 1)
    def _():
        pltpu.sync_copy(col_hbm.at[pl.ds(base, EPT)], idx_v)

    pltpu.sync_copy(ew_hbm.at[pl.ds(base, EPT)], w_v)

    @pl.loop(0, NP, step=16)
    def _(i):
        deg_v[pl.ds(i, 16)] = jnp.zeros((16,), jnp.float32)

    @pl.loop(0, EPT, step=16)
    def _(j):
        plsc.addupdate_scatter(deg_v, [idx_v[pl.ds(j, 16)]], w_v[pl.ds(j, 16)])

    pltpu.sync_copy(deg_v, stage_sh.at[sid])
    plsc.subcore_barrier()
    pltpu.sync_copy(stage_sh.at[:, pl.ds(sid * RED, RED)], red_v)

    @pl.loop(0, RED, step=16)
    def _(i):
        acc = red_v[0, pl.ds(i, 16)]
        for r in range(1, NS):
            acc = acc + red_v[r, pl.ds(i, 16)]
        res_v[pl.ds(i, 16)] = acc

    @pl.when(cid == 0)
    def _():
        pltpu.sync_copy(res_v, dout_hbm.at[pl.ds(sid * RED, RED)])

    @pl.when(cid == 1)
    def _():
        pltpu.sync_copy(res_v, din_hbm.at[pl.ds(sid * RED, RED)])


_deg_kernel = functools.partial(
    pl.kernel,
    out_type=(jax.ShapeDtypeStruct((NP,), jnp.float32),
              jax.ShapeDtypeStruct((NP,), jnp.float32)),
    mesh=_mesh,
    scratch_types=[
        pltpu.VMEM((EPT,), jnp.int32),
        pltpu.VMEM((EPT,), jnp.float32),
        pltpu.VMEM((NP,), jnp.float32),
        pltpu.VMEM((RED,), jnp.float32),
        pltpu.VMEM((NS, RED), jnp.float32),
        pltpu.VMEM_SHARED((NS, NP), jnp.float32),
    ],
    compiler_params=_sc_params,
)(_deg_body)


NCH_STD = 160
NCH_LAST = 100
ER = E // CH
CPI = 8


def _prop_half(rc_hbm, tbl_hbm, acc_sh, rc_i, gbufs, isems, gsems, ssems,
               r0, nch):
    def i_start(j, p):
        pltpu.async_copy(rc_hbm.at[r0 + j], rc_i.at[p], isems[p])

    def i_wait(p):
        pltpu.make_async_copy(rc_hbm.at[r0], rc_i.at[p], isems[p]).wait()

    def g_start(p):
        pltpu.async_copy(tbl_hbm.at[rc_i.at[p, 0]], gbufs[p], gsems[p])

    def g_wait(p):
        pltpu.make_async_copy(tbl_hbm.at[rc_i.at[p, 0]], gbufs[p],
                              gsems[p]).wait()

    def s_start(p):
        pltpu.async_copy(gbufs[p], acc_sh.at[rc_i.at[p, 1]], ssems[p],
                         add=True)

    def s_wait(p):
        pltpu.make_async_copy(gbufs[p], acc_sh.at[rc_i.at[p, 1]],
                              ssems[p]).wait()

    def pair(a, prefetch, terminal):
        g_wait(0)
        s_start(0)
        g_wait(1)
        s_start(1)
        if prefetch:
            s_wait(0)
            i_start(a + 2, 0)
            s_wait(1)
            i_start(a + 3, 1)
            i_wait(0)
            g_start(0)
            i_wait(1)
            g_start(1)
        if terminal:
            s_wait(0)
            s_wait(1)

    i_start(0, 0)
    i_start(1, 1)
    i_wait(0)
    g_start(0)
    i_wait(1)
    g_start(1)

    full = (nch - CPI) // CPI

    @pl.loop(0, full)
    def _(k):
        c = CPI * k
        for t in range(CPI // 2):
            pair(c + 2 * t, True, False)

    a = full * CPI
    while a < nch:
        pair(a, a + 2 < nch, a + 2 >= nch)
        a += 2


def _prop_body(rc_hbm, tout_hbm, tin_hbm, pout_hbm, pin_hbm,
               rc_i, g_0, g_1, acc_sh,
               sem_i0, sem_i1, sem_g0, sem_g1, sem_s0, sem_s1):
    cid = lax.axis_index("c")
    sid = lax.axis_index("s")
    gbufs = (g_0, g_1)
    isems = (sem_i0, sem_i1)
    gsems = (sem_g0, sem_g1)
    ssems = (sem_s0, sem_s1)

    @pl.loop(0, ZB)
    def _(i):
        @pl.loop(0, F2, step=16)
        def _(k2):
            g_0[i, pl.ds(k2, 16)] = jnp.zeros((16,), jnp.float32)

    for k in range(ROWS_PT // ZB):
        pltpu.sync_copy(g_0, acc_sh.at[pl.ds(sid * ROWS_PT + k * ZB, ZB)])
    plsc.subcore_barrier()

    def run(tbl_hbm):
        @pl.when(sid < NS - 1)
        def _():
            _prop_half(rc_hbm, tbl_hbm, acc_sh, rc_i, gbufs, isems, gsems,
                       ssems, sid * NCH_STD, NCH_STD)

        @pl.when(sid == NS - 1)
        def _():
            _prop_half(rc_hbm, tbl_hbm, acc_sh, rc_i, gbufs, isems, gsems,
                       ssems, (NS - 1) * NCH_STD, NCH_LAST)

    @pl.when(cid == 0)
    def _():
        run(tout_hbm)

    @pl.when(cid == 1)
    def _():
        run(tin_hbm)

    plsc.subcore_barrier()

    @pl.when(cid == 0)
    def _():
        pltpu.sync_copy(acc_sh.at[pl.ds(sid * ROWS_PT, ROWS_PT)],
                        pout_hbm.at[pl.ds(sid * ROWS_PT, ROWS_PT)])

    @pl.when(cid == 1)
    def _():
        pltpu.sync_copy(acc_sh.at[pl.ds(sid * ROWS_PT, ROWS_PT)],
                        pin_hbm.at[pl.ds(sid * ROWS_PT, ROWS_PT)])


_prop_kernel = functools.partial(
    pl.kernel,
    out_type=(jax.ShapeDtypeStruct((NP, F2), jnp.float32),
              jax.ShapeDtypeStruct((NP, F2), jnp.float32)),
    mesh=_mesh,
    scratch_types=[
        pltpu.VMEM((2, 2, CH), jnp.int32),
        pltpu.VMEM((CH, F2), jnp.float32),
        pltpu.VMEM((CH, F2), jnp.float32),
        pltpu.VMEM_SHARED((NP, F2), jnp.float32),
    ] + [pltpu.SemaphoreType.DMA] * 6,
    compiler_params=_sc_params,
)(_prop_body)


BLK = 1000


def _mm_body(x_ref, w_ref, d_ref, to_ref, ti_ref, b0_ref):
    u = jnp.dot(x_ref[...], w_ref[...], preferred_element_type=jnp.float32)
    d = d_ref[...]
    r = jnp.where(d > 0, 1.0 / d, 0.0)
    to_ref[...] = u[:, :F2] * r
    ti_ref[...] = u[:, F2:2 * F2]
    b0_ref[...] = u[:, 2 * F2:]


def _matmul_tc(x, w, d):
    m, k = x.shape
    spec = pl.BlockSpec((BLK, F2), lambda i: (i, 0))
    return pl.pallas_call(
        _mm_body,
        grid=(m // BLK,),
        in_specs=[pl.BlockSpec((BLK, k), lambda i: (i, 0)),
                  pl.BlockSpec((k, 3 * F2), lambda i: (0, 0)),
                  pl.BlockSpec((BLK, 1), lambda i: (i, 0))],
        out_specs=(spec, spec, spec),
        out_shape=(jax.ShapeDtypeStruct((m, F2), jnp.float32),
                   jax.ShapeDtypeStruct((m, F2), jnp.float32),
                   jax.ShapeDtypeStruct((m, F2), jnp.float32)),
    )(x, w, d)


def _combine_body(b0_ref, po_ref, pi_ref, din_ref, bz_ref, bh_ref, lw_ref,
                  lb_ref, o_ref):
    d = din_ref[...]
    r = jnp.where(d > 0, 1.0 / d, 0.0)
    b0 = b0_ref[...]
    po = po_ref[...]
    pi = pi_ref[...] * r
    lz = b0[:, :F_OUT] + po[:, :F_OUT] + pi[:, :F_OUT] + bz_ref[...]
    lh = b0[:, F_OUT:] + po[:, F_OUT:] + pi[:, F_OUT:] + bh_ref[...]
    z = jax.nn.sigmoid(lz)
    ht = jnp.tanh(lh)
    h = jax.nn.relu((1.0 - z) * ht)
    o_ref[...] = (jnp.dot(h, lw_ref[...], preferred_element_type=jnp.float32)
                  + lb_ref[...])


def _combine_tc(b0, po, pi, din, bz2, bh2, lw, lb2):
    m = b0.shape[0]
    return pl.pallas_call(
        _combine_body,
        grid=(m // BLK,),
        in_specs=[pl.BlockSpec((BLK, F2), lambda i: (i, 0)),
                  pl.BlockSpec((BLK, F2), lambda i: (i, 0)),
                  pl.BlockSpec((BLK, F2), lambda i: (i, 0)),
                  pl.BlockSpec((BLK, 1), lambda i: (i, 0)),
                  pl.BlockSpec((1, F_OUT), lambda i: (0, 0)),
                  pl.BlockSpec((1, F_OUT), lambda i: (0, 0)),
                  pl.BlockSpec((F_OUT, 1), lambda i: (0, 0)),
                  pl.BlockSpec((1, 1), lambda i: (0, 0))],
        out_specs=pl.BlockSpec((BLK, 1), lambda i: (i, 0)),
        out_shape=jax.ShapeDtypeStruct((m, 1), jnp.float32),
    )(b0, po, pi, din, bz2, bh2, lw, lb2)


def kernel(x, edge_index, edge_weight, h_, c, Wz, bz, Wr, br, Wh, bh,
           lin_w, lin_b):
    wo = jnp.concatenate([Wz[0, 1, :F_IN], Wh[0, 1, :F_IN]], axis=1)
    wi = jnp.concatenate([Wz[1, 1, :F_IN], Wh[1, 1, :F_IN]], axis=1)
    w0 = jnp.concatenate([Wz[0, 0, :F_IN] + Wz[1, 0, :F_IN],
                          Wh[0, 0, :F_IN] + Wh[1, 0, :F_IN]], axis=1)
    wall = jnp.concatenate([wo, wi, w0], axis=1)

    row = edge_index[0]
    col = edge_index[1]
    deg_out_p, deg_in_p = _deg_kernel(row, col, edge_weight)
    t_out, t_in, b0 = _matmul_tc(x, wall, deg_out_p[:N].reshape(N, 1))
    rc = jnp.stack([row.reshape(ER, CH), col.reshape(ER, CH)], axis=1)
    p_out_p, p_in_p = _prop_kernel(rc, t_out, t_in)
    p_out = p_out_p[:N]
    p_in = p_in_p[:N]
    out = _combine_tc(b0, p_out, p_in,
                      deg_in_p[:N].reshape(N, 1),
                      bz.reshape(1, F_OUT), bh.reshape(1, F_OUT),
                      lin_w, lin_b.reshape(1, 1))
    return out


# speedup vs baseline: 1.0405x; 1.0229x over previous
import dataclasses
import functools

import jax
import jax.numpy as jnp
from jax import lax
from jax.experimental import pallas as pl
from jax.experimental.pallas import tpu as pltpu
from jax.experimental.pallas import tpu_sc as plsc

N = 10000
E = 320000
F_IN = 128
F_OUT = 64
F2 = 2 * F_OUT

NC = 2
NS = 16
NP = 10240
RED = NP // NS
EPT = E // NS
CH = 128
ROWS_PT = NP // NS
ZB = 128

_mesh = plsc.VectorSubcoreMesh(core_axis_name="c", subcore_axis_name="s")

_sc_params = pltpu.CompilerParams()
if "needs_layout_passes" in pltpu.CompilerParams.__dataclass_fields__:
    _sc_params = dataclasses.replace(_sc_params, needs_layout_passes=False)


def _deg_body(row_hbm, col_hbm, ew_hbm, dout_hbm, din_hbm,
              idx_v, w_v, deg_v, res_v, red_v, stage_sh):
    cid = lax.axis_index("c")
    sid = lax.axis_index("s")
    base = sid * EPT

    @pl.when(cid == 0)
    def _():
        pltpu.sync_copy(row_hbm.at[pl.ds(base, EPT)], idx_v)

    @pl.when(cid == 1)
    def _():
        pltpu.sync_copy(col_hbm.at[pl.ds(base, EPT)], idx_v)

    pltpu.sync_copy(ew_hbm.at[pl.ds(base, EPT)], w_v)

    @pl.loop(0, NP, step=16)
    def _(i):
        deg_v[pl.ds(i, 16)] = jnp.zeros((16,), jnp.float32)

    @pl.loop(0, EPT, step=16)
    def _(j):
        plsc.addupdate_scatter(deg_v, [idx_v[pl.ds(j, 16)]], w_v[pl.ds(j, 16)])

    pltpu.sync_copy(deg_v, stage_sh.at[sid])
    plsc.subcore_barrier()
    pltpu.sync_copy(stage_sh.at[:, pl.ds(sid * RED, RED)], red_v)

    @pl.loop(0, RED, step=16)
    def _(i):
        acc = red_v[0, pl.ds(i, 16)]
        for r in range(1, NS):
            acc = acc + red_v[r, pl.ds(i, 16)]
        res_v[pl.ds(i, 16)] = acc

    @pl.when(cid == 0)
    def _():
        pltpu.sync_copy(res_v, dout_hbm.at[pl.ds(sid * RED, RED)])

    @pl.when(cid == 1)
    def _():
        pltpu.sync_copy(res_v, din_hbm.at[pl.ds(sid * RED, RED)])


_deg_kernel = functools.partial(
    pl.kernel,
    out_type=(jax.ShapeDtypeStruct((NP,), jnp.float32),
              jax.ShapeDtypeStruct((NP,), jnp.float32)),
    mesh=_mesh,
    scratch_types=[
        pltpu.VMEM((EPT,), jnp.int32),
        pltpu.VMEM((EPT,), jnp.float32),
        pltpu.VMEM((NP,), jnp.float32),
        pltpu.VMEM((RED,), jnp.float32),
        pltpu.VMEM((NS, RED), jnp.float32),
        pltpu.VMEM_SHARED((NS, NP), jnp.float32),
    ],
    compiler_params=_sc_params,
)(_deg_body)


NCH_STD = 160
NCH_LAST = 100
ER = E // CH
CPI = 8


def _prop_half(rc_hbm, tbl_hbm, acc_sh, rc_i, gbufs, isems, gsems, ssems,
               r0, nch):
    def i_start(j, p):
        pltpu.async_copy(rc_hbm.at[r0 + j], rc_i.at[p], isems[p])

    def i_wait(p):
        pltpu.make_async_copy(rc_hbm.at[r0], rc_i.at[p], isems[p]).wait()

    def g_start(p):
        pltpu.async_copy(tbl_hbm.at[rc_i.at[p, 0]], gbufs[p], gsems[p])

    def g_wait(p):
        pltpu.make_async_copy(tbl_hbm.at[rc_i.at[p, 0]], gbufs[p],
                              gsems[p]).wait()

    def s_start(p):
        pltpu.async_copy(gbufs[p], acc_sh.at[rc_i.at[p, 1]], ssems[p],
                         add=True)

    def s_wait(p):
        pltpu.make_async_copy(gbufs[p], acc_sh.at[rc_i.at[p, 1]],
                              ssems[p]).wait()

    def pair(a, prefetch, terminal):
        g_wait(0)
        s_start(0)
        g_wait(1)
        s_start(1)
        if prefetch:
            s_wait(0)
            i_start(a + 2, 0)
            s_wait(1)
            i_start(a + 3, 1)
            i_wait(0)
            g_start(0)
            i_wait(1)
            g_start(1)
        if terminal:
            s_wait(0)
            s_wait(1)

    i_start(0, 0)
    i_start(1, 1)
    i_wait(0)
    g_start(0)
    i_wait(1)
    g_start(1)

    full = (nch - CPI) // CPI

    @pl.loop(0, full)
    def _(k):
        c = CPI * k
        for t in range(CPI // 2):
            pair(c + 2 * t, True, False)

    a = full * CPI
    while a < nch:
        pair(a, a + 2 < nch, a + 2 >= nch)
        a += 2


def _prop_body(rc_hbm, tout_hbm, tin_hbm, pout_hbm, pin_hbm,
               rc_i, g_0, g_1, acc_sh,
               sem_i0, sem_i1, sem_g0, sem_g1, sem_s0, sem_s1):
    cid = lax.axis_index("c")
    sid = lax.axis_index("s")
    gbufs = (g_0, g_1)
    isems = (sem_i0, sem_i1)
    gsems = (sem_g0, sem_g1)
    ssems = (sem_s0, sem_s1)

    @pl.loop(0, ZB)
    def _(i):
        @pl.loop(0, F2, step=16)
        def _(k2):
            g_0[i, pl.ds(k2, 16)] = jnp.zeros((16,), jnp.float32)

    for k in range(ROWS_PT // ZB):
        pltpu.sync_copy(g_0, acc_sh.at[pl.ds(sid * ROWS_PT + k * ZB, ZB)])
    plsc.subcore_barrier()

    def run(tbl_hbm):
        @pl.when(sid < NS - 1)
        def _():
            _prop_half(rc_hbm, tbl_hbm, acc_sh, rc_i, gbufs, isems, gsems,
                       ssems, sid * NCH_STD, NCH_STD)

        @pl.when(sid == NS - 1)
        def _():
            _prop_half(rc_hbm, tbl_hbm, acc_sh, rc_i, gbufs, isems, gsems,
                       ssems, (NS - 1) * NCH_STD, NCH_LAST)

    @pl.when(cid == 0)
    def _():
        run(tout_hbm)

    @pl.when(cid == 1)
    def _():
        run(tin_hbm)

    plsc.subcore_barrier()

    @pl.when(cid == 0)
    def _():
        pltpu.sync_copy(acc_sh.at[pl.ds(sid * ROWS_PT, ROWS_PT)],
                        pout_hbm.at[pl.ds(sid * ROWS_PT, ROWS_PT)])

    @pl.when(cid == 1)
    def _():
        pltpu.sync_copy(acc_sh.at[pl.ds(sid * ROWS_PT, ROWS_PT)],
                        pin_hbm.at[pl.ds(sid * ROWS_PT, ROWS_PT)])


_prop_kernel = functools.partial(
    pl.kernel,
    out_type=(jax.ShapeDtypeStruct((NP, F2), jnp.float32),
              jax.ShapeDtypeStruct((NP, F2), jnp.float32)),
    mesh=_mesh,
    scratch_types=[
        pltpu.VMEM((2, 2, CH), jnp.int32),
        pltpu.VMEM((CH, F2), jnp.float32),
        pltpu.VMEM((CH, F2), jnp.float32),
        pltpu.VMEM_SHARED((NP, F2), jnp.float32),
    ] + [pltpu.SemaphoreType.DMA] * 6,
    compiler_params=_sc_params,
)(_prop_body)


BLK = 1000


def _mm_body(x_ref, w_ref, d_ref, to_ref, ti_ref, b0_ref):
    u = jnp.dot(x_ref[...], w_ref[...], preferred_element_type=jnp.float32)
    d = d_ref[...]
    r = jnp.where(d > 0, 1.0 / d, 0.0)
    to_ref[...] = u[:, :F2] * r
    ti_ref[...] = u[:, F2:2 * F2]
    b0_ref[...] = u[:, 2 * F2:]


def _matmul_tc(x, w, d):
    m, k = x.shape
    spec = pl.BlockSpec((BLK, F2), lambda i: (i, 0))
    return pl.pallas_call(
        _mm_body,
        grid=(m // BLK,),
        in_specs=[pl.BlockSpec((BLK, k), lambda i: (i, 0)),
                  pl.BlockSpec((k, 3 * F2), lambda i: (0, 0)),
                  pl.BlockSpec((BLK, 1), lambda i: (i, 0))],
        out_specs=(spec, spec, spec),
        out_shape=(jax.ShapeDtypeStruct((m, F2), jnp.float32),
                   jax.ShapeDtypeStruct((m, F2), jnp.float32),
                   jax.ShapeDtypeStruct((m, F2), jnp.float32)),
    )(x, w, d)


def _combine_body(b0_ref, po_ref, pi_ref, din_ref, bz_ref, bh_ref, lw_ref,
                  lb_ref, o_ref):
    d = din_ref[...]
    r = jnp.where(d > 0, 1.0 / d, 0.0)
    b0 = b0_ref[...]
    po = po_ref[...]
    pi = pi_ref[...] * r
    lz = b0[:, :F_OUT] + po[:, :F_OUT] + pi[:, :F_OUT] + bz_ref[...]
    lh = b0[:, F_OUT:] + po[:, F_OUT:] + pi[:, F_OUT:] + bh_ref[...]
    z = jax.nn.sigmoid(lz)
    ht = jnp.tanh(lh)
    h = jax.nn.relu((1.0 - z) * ht)
    o_ref[...] = (jnp.dot(h, lw_ref[...], preferred_element_type=jnp.float32)
                  + lb_ref[...])


def _combine_tc(b0, po, pi, din, bz2, bh2, lw, lb2):
    return pl.pallas_call(
        _combine_body,
        grid=(N // BLK,),
        in_specs=[pl.BlockSpec((BLK, F2), lambda i: (i, 0)),
                  pl.BlockSpec((BLK, F2), lambda i: (i, 0)),
                  pl.BlockSpec((BLK, F2), lambda i: (i, 0)),
                  pl.BlockSpec((BLK, 1), lambda i: (i, 0)),
                  pl.BlockSpec((1, F_OUT), lambda i: (0, 0)),
                  pl.BlockSpec((1, F_OUT), lambda i: (0, 0)),
                  pl.BlockSpec((F_OUT, 1), lambda i: (0, 0)),
                  pl.BlockSpec((1, 1), lambda i: (0, 0))],
        out_specs=pl.BlockSpec((BLK, 1), lambda i: (i, 0)),
        out_shape=jax.ShapeDtypeStruct((N, 1), jnp.float32),
    )(b0, po, pi, din, bz2, bh2, lw, lb2)


def kernel(x, edge_index, edge_weight, h_, c, Wz, bz, Wr, br, Wh, bh,
           lin_w, lin_b):
    wo = jnp.concatenate([Wz[0, 1, :F_IN], Wh[0, 1, :F_IN]], axis=1)
    wi = jnp.concatenate([Wz[1, 1, :F_IN], Wh[1, 1, :F_IN]], axis=1)
    w0 = jnp.concatenate([Wz[0, 0, :F_IN] + Wz[1, 0, :F_IN],
                          Wh[0, 0, :F_IN] + Wh[1, 0, :F_IN]], axis=1)
    wall = jnp.concatenate([wo, wi, w0], axis=1)

    row = edge_index[0]
    col = edge_index[1]
    deg_out_p, deg_in_p = _deg_kernel(row, col, edge_weight)
    t_out, t_in, b0 = _matmul_tc(x, wall, deg_out_p.reshape(NP, 1))
    rc = jnp.stack([row.reshape(ER, CH), col.reshape(ER, CH)], axis=1)
    p_out_p, p_in_p = _prop_kernel(rc, t_out, t_in)
    out = _combine_tc(b0, p_out_p, p_in_p,
                      deg_in_p.reshape(NP, 1),
                      bz.reshape(1, F_OUT), bh.reshape(1, F_OUT),
                      lin_w, lin_b.reshape(1, 1))
    return out
